# baseline (speedup 1.0000x reference)
import functools

import jax
import jax.numpy as jnp
from jax import lax
from jax.experimental import pallas as pl
from jax.experimental.pallas import tpu as pltpu
from jax.experimental.pallas import tpu_sc as plsc

L = 16
KEY_BITS = 10
S = 5
SB = 1024


def _sort_tile_kernel(EPG, B, sd_hbm, val_hbm, splits_hbm,
                      outs_hbm, outd_hbm, outv_hbm,
                      sd_v, val_v, q_perm, v_perm, out_d,
                      hists_all, splits_v, *histd):
    NBINS = histd[0].shape[0]
    SLEN = EPG // S
    SNV = SLEN // L

    wid = lax.axis_index("s") * 2 + lax.axis_index("c")
    g = wid
    MASK = jnp.int32((1 << KEY_BITS) - 1)

    @pl.when(wid < B)
    def _body():
        base_e = g * EPG
        pltpu.sync_copy(sd_hbm.at[pl.ds(base_e, EPG)], sd_v)
        pltpu.sync_copy(val_hbm.at[pl.ds(base_e, EPG)], val_v)
        pltpu.sync_copy(splits_hbm, splits_v)

        zeros = jnp.zeros((L,), jnp.int32)

        def zero_bins(i, c):
            for s in range(S):
                histd[s][pl.ds(i * L, L)] = zeros
                hists_all[pl.ds(s * SB + i * L, L)] = zeros
            return c

        lax.fori_loop(0, SB // L, zero_bins, 0)

        def hist1_body(i, c):
            for s in range(S):
                q = sd_v[pl.ds((s * SNV + i) * L, L)]
                kd = q & MASK
                cnt, last = plsc.scan_count(kd)
                cur = plsc.load_gather(histd[s], [kd])
                plsc.store_scatter(histd[s], [kd], cur + cnt, mask=last)
            return c

        lax.fori_loop(0, SNV, hist1_body, 0)

        def prefix1_body(b, carry):
            hs = [histd[s][pl.ds(b * L, L)] for s in range(S)]
            total = hs[0]
            for s in range(1, S):
                total = total + hs[s]
            inc = plsc.cumsum(total)
            acc = inc - total + carry
            for s in range(S):
                histd[s][pl.ds(b * L, L)] = acc
                acc = acc + hs[s]
            return carry + jnp.sum(total)

        lax.fori_loop(0, NBINS // L, prefix1_body, jnp.int32(0))

        def pass1_body(i, c):
            for s in range(S):
                ii = s * SNV + i
                q = sd_v[pl.ds(ii * L, L)]
                v = val_v[pl.ds(ii * L, L)]
                kd = q & MASK
                cnt, last = plsc.scan_count(kd)
                cur = plsc.load_gather(histd[s], [kd])
                plsc.store_scatter(histd[s], [kd], cur + cnt, mask=last)
                pos = cur + cnt - 1
                plsc.store_scatter(q_perm, [pos], q)
                plsc.store_scatter(v_perm, [pos], v)
                ks = lax.shift_right_logical(q, KEY_BITS)
                kc = ((pos // SLEN) << KEY_BITS) | ks
                cnt2, last2 = plsc.scan_count(kc)
                cur2 = plsc.load_gather(hists_all, [kc])
                plsc.store_scatter(hists_all, [kc], cur2 + cnt2, mask=last2)
            return c

        lax.fori_loop(0, SNV, pass1_body, 0)

        def prefix2_body(b, carry):
            hs = [hists_all[pl.ds(s * SB + b * L, L)] for s in range(S)]
            total = hs[0]
            for s in range(1, S):
                total = total + hs[s]
            inc = plsc.cumsum(total)
            acc = inc - total + carry
            for s in range(S):
                hists_all[pl.ds(s * SB + b * L, L)] = acc
                acc = acc + hs[s]
            return carry + jnp.sum(total)

        lax.fori_loop(0, SB // L, prefix2_body, jnp.int32(0))

        out_s = sd_v
        out_v = val_v
        nbase = plsc.load_gather(splits_v, [jnp.full((L,), g, jnp.int32)])

        def pass2_body(i, c):
            for s in range(S):
                ii = s * SNV + i
                q = q_perm[pl.ds(ii * L, L)]
                v = v_perm[pl.ds(ii * L, L)]
                ks = lax.shift_right_logical(q, KEY_BITS)
                kc = ks + s * SB
                cnt, last = plsc.scan_count(ks)
                cur = plsc.load_gather(hists_all, [kc])
                plsc.store_scatter(hists_all, [kc], cur + cnt, mask=last)
                pos = cur + cnt - 1
                plsc.store_scatter(out_s, [pos], ks + nbase)
                plsc.store_scatter(out_d, [pos], (q & MASK) + nbase)
                plsc.store_scatter(out_v, [pos], v)
            return c

        lax.fori_loop(0, SNV, pass2_body, 0)

        pltpu.sync_copy(out_s, outs_hbm.at[pl.ds(base_e, EPG)])
        pltpu.sync_copy(out_d, outd_hbm.at[pl.ds(base_e, EPG)])
        pltpu.sync_copy(out_v, outv_hbm.at[pl.ds(base_e, EPG)])


def _make_sorter(E, B, NPG):
    EPG = E // B
    NBINS = ((NPG + L - 1) // L) * L
    mesh = plsc.VectorSubcoreMesh(core_axis_name="c", subcore_axis_name="s")
    i32 = jnp.int32
    f32 = jnp.float32
    return pl.kernel(
        functools.partial(_sort_tile_kernel, EPG, B),
        out_type=(
            jax.ShapeDtypeStruct((E,), i32),
            jax.ShapeDtypeStruct((E,), i32),
            jax.ShapeDtypeStruct((E,), f32),
        ),
        mesh=mesh,
        compiler_params=pltpu.CompilerParams(needs_layout_passes=False),
        scratch_types=[
            pltpu.VMEM((EPG,), i32),
            pltpu.VMEM((EPG,), f32),
            pltpu.VMEM((EPG,), i32),
            pltpu.VMEM((EPG,), f32),
            pltpu.VMEM((EPG,), i32),
            pltpu.VMEM((S * SB,), i32),
            pltpu.VMEM((L,), i32),
        ] + [pltpu.VMEM((NBINS,), i32) for _ in range(S)],
    )


def kernel(node_values, node_row_splits, edge_index, edge_row_lengths, edge_feat):
    E = edge_index.shape[0]
    B = node_row_splits.shape[0] - 1
    n = node_values.shape[0]
    NPG = n // B

    ei = edge_index.astype(jnp.int32)
    sd = (ei[:, 0] << KEY_BITS) | ei[:, 1]
    val = edge_feat[:, 0]
    splits = node_row_splits[:B].astype(jnp.int32)

    sorter = _make_sorter(E, B, NPG)
    out_s, out_d, out_v = sorter(sd, val, splits)

    indexlist = jnp.stack([out_s, out_d], axis=1).astype(jnp.int64)
    dense_shape = jnp.array([n, n], dtype=jnp.int64)
    return indexlist, out_v, dense_shape


# speedup vs baseline: 1.2858x; 1.2858x over previous
import functools

import jax
import jax.numpy as jnp
from jax import lax
from jax.experimental import pallas as pl
from jax.experimental.pallas import tpu as pltpu
from jax.experimental.pallas import tpu_sc as plsc

L = 16
KEY_BITS = 10
S = 5
SB = 1024


def _sort_tile_kernel(EPG, B, sd_hbm, val_hbm, splits_hbm,
                      outs_hbm, outd_hbm, outv_hbm,
                      sd_v, val_v, q_perm, v_perm, out_d,
                      splits_v, *bins):
    histd = bins[:S]
    hists = bins[S:]
    NBINS = histd[0].shape[0]
    SLEN = EPG // S
    SNV = SLEN // L

    wid = lax.axis_index("s") * 2 + lax.axis_index("c")
    g = wid
    MASK = jnp.int32((1 << KEY_BITS) - 1)

    @pl.when(wid < B)
    def _body():
        base_e = g * EPG
        pltpu.sync_copy(sd_hbm.at[pl.ds(base_e, EPG)], sd_v)
        pltpu.sync_copy(val_hbm.at[pl.ds(base_e, EPG)], val_v)
        pltpu.sync_copy(splits_hbm, splits_v)

        zeros = jnp.zeros((L,), jnp.int32)

        def zero_bins(i, c):
            for s in range(S):
                histd[s][pl.ds(i * L, L)] = zeros
                hists[s][pl.ds(i * L, L)] = zeros
            return c

        lax.fori_loop(0, NBINS // L, zero_bins, 0)

        def hist1_body(i, c):
            for s in range(S):
                q = sd_v[pl.ds((s * SNV + i) * L, L)]
                kd = q & MASK
                cnt, last = plsc.scan_count(kd)
                cur = plsc.load_gather(histd[s], [kd])
                plsc.store_scatter(histd[s], [kd], cur + cnt, mask=last)
            return c

        lax.fori_loop(0, SNV, hist1_body, 0)

        def prefix1_body(b, carry):
            hs = [histd[s][pl.ds(b * L, L)] for s in range(S)]
            total = hs[0]
            for s in range(1, S):
                total = total + hs[s]
            inc = plsc.cumsum(total)
            acc = inc - total + carry
            for s in range(S):
                histd[s][pl.ds(b * L, L)] = acc
                acc = acc + hs[s]
            return carry + jnp.sum(total)

        lax.fori_loop(0, NBINS // L, prefix1_body, jnp.int32(0))

        def pass1_body(i, c):
            for s in range(S):
                ii = s * SNV + i
                q = sd_v[pl.ds(ii * L, L)]
                v = val_v[pl.ds(ii * L, L)]
                kd = q & MASK
                cnt, last = plsc.scan_count(kd)
                cur = plsc.load_gather(histd[s], [kd])
                plsc.store_scatter(histd[s], [kd], cur + cnt, mask=last)
                pos = cur + cnt - 1
                plsc.store_scatter(q_perm, [pos], q)
                plsc.store_scatter(v_perm, [pos], v)
            return c

        lax.fori_loop(0, SNV, pass1_body, 0)

        def hist2_body(i, c):
            for s in range(S):
                q = q_perm[pl.ds((s * SNV + i) * L, L)]
                ks = lax.shift_right_logical(q, KEY_BITS)
                cnt, last = plsc.scan_count(ks)
                cur = plsc.load_gather(hists[s], [ks])
                plsc.store_scatter(hists[s], [ks], cur + cnt, mask=last)
            return c

        lax.fori_loop(0, SNV, hist2_body, 0)

        def prefix2_body(b, carry):
            hs = [hists[s][pl.ds(b * L, L)] for s in range(S)]
            total = hs[0]
            for s in range(1, S):
                total = total + hs[s]
            inc = plsc.cumsum(total)
            acc = inc - total + carry
            for s in range(S):
                hists[s][pl.ds(b * L, L)] = acc
                acc = acc + hs[s]
            return carry + jnp.sum(total)

        lax.fori_loop(0, NBINS // L, prefix2_body, jnp.int32(0))

        out_s = sd_v
        out_v = val_v
        nbase = plsc.load_gather(splits_v, [jnp.full((L,), g, jnp.int32)])

        def pass2_body(i, c):
            for s in range(S):
                ii = s * SNV + i
                q = q_perm[pl.ds(ii * L, L)]
                v = v_perm[pl.ds(ii * L, L)]
                ks = lax.shift_right_logical(q, KEY_BITS)
                cnt, last = plsc.scan_count(ks)
                cur = plsc.load_gather(hists[s], [ks])
                plsc.store_scatter(hists[s], [ks], cur + cnt, mask=last)
                pos = cur + cnt - 1
                plsc.store_scatter(out_s, [pos], ks + nbase)
                plsc.store_scatter(out_d, [pos], (q & MASK) + nbase)
                plsc.store_scatter(out_v, [pos], v)
            return c

        lax.fori_loop(0, SNV, pass2_body, 0)

        pltpu.sync_copy(out_s, outs_hbm.at[pl.ds(base_e, EPG)])
        pltpu.sync_copy(out_d, outd_hbm.at[pl.ds(base_e, EPG)])
        pltpu.sync_copy(out_v, outv_hbm.at[pl.ds(base_e, EPG)])


def _make_sorter(E, B, NPG):
    EPG = E // B
    NBINS = ((NPG + L - 1) // L) * L
    mesh = plsc.VectorSubcoreMesh(core_axis_name="c", subcore_axis_name="s")
    i32 = jnp.int32
    f32 = jnp.float32
    return pl.kernel(
        functools.partial(_sort_tile_kernel, EPG, B),
        out_type=(
            jax.ShapeDtypeStruct((E,), i32),
            jax.ShapeDtypeStruct((E,), i32),
            jax.ShapeDtypeStruct((E,), f32),
        ),
        mesh=mesh,
        compiler_params=pltpu.CompilerParams(needs_layout_passes=False),
        scratch_types=[
            pltpu.VMEM((EPG,), i32),
            pltpu.VMEM((EPG,), f32),
            pltpu.VMEM((EPG,), i32),
            pltpu.VMEM((EPG,), f32),
            pltpu.VMEM((EPG,), i32),
            pltpu.VMEM((L,), i32),
        ] + [pltpu.VMEM((NBINS,), i32) for _ in range(2 * S)],
    )


def kernel(node_values, node_row_splits, edge_index, edge_row_lengths, edge_feat):
    E = edge_index.shape[0]
    B = node_row_splits.shape[0] - 1
    n = node_values.shape[0]
    NPG = n // B

    ei = edge_index.astype(jnp.int32)
    sd = (ei[:, 0] << KEY_BITS) | ei[:, 1]
    val = edge_feat[:, 0]
    splits = node_row_splits[:B].astype(jnp.int32)

    sorter = _make_sorter(E, B, NPG)
    out_s, out_d, out_v = sorter(sd, val, splits)

    indexlist = jnp.stack([out_s, out_d], axis=1).astype(jnp.int64)
    dense_shape = jnp.array([n, n], dtype=jnp.int64)
    return indexlist, out_v, dense_shape
